# baseline (speedup 1.0000x reference)
import functools

import jax
import jax.numpy as jnp
from jax import lax
from jax.experimental import pallas as pl
from jax.experimental.pallas import tpu as pltpu
from jax.experimental.pallas import tpu_sc as plsc

N = 10000
D_IN = 128
H = 64
C = 2
E = 320000

NP = 10240
ROWS_PER_TILE = NP // 16
CHUNK = 128
BATCH = 4
SUPER = BATCH * CHUNK
CHUNKS_PER_W = 160
SCH = CHUNKS_PER_W // BATCH
EPW = CHUNKS_PER_W * CHUNK
EPAD = 16 * EPW
CW = 16

_f32 = jnp.float32
_bf16 = jnp.bfloat16
_HIGH = jax.lax.Precision.HIGHEST


@functools.cache
def _get_mesh():
    return plsc.VectorSubcoreMesh(core_axis_name="c", subcore_axis_name="s")


def _sc_body(with_counts, ycat, src_all, dst_all, agg_out, cnt_out,
             src_v, dst_v, rows_a, rows_b, ones_v, acc, cnt,
             sem_a, sem_b, sem_c):
    rel = lax.axis_index("c")
    sid = lax.axis_index("s")
    base = sid * ROWS_PER_TILE

    def zrow(i, _):
        for c in range(H // 32):
            rows_a[i, pl.ds(c * 32, 32)] = jnp.zeros((32,), _bf16)
        return 0
    lax.fori_loop(0, SUPER, zrow, 0)
    zparts = [(0, SUPER), (SUPER, ROWS_PER_TILE - SUPER)]
    for off, sz in zparts:
        pltpu.async_copy(rows_a.at[pl.ds(0, sz)],
                         acc.at[pl.ds(base + off, sz)], sem_c)
    if with_counts:
        def crow(i, _):
            ones_v[i, :] = jnp.zeros((CW,), _f32)
            return 0
        lax.fori_loop(0, SUPER, crow, 0)
        for off, sz in zparts:
            pltpu.async_copy(ones_v.at[pl.ds(0, sz)],
                             cnt.at[pl.ds(base + off, sz)], sem_c)
    for off, sz in zparts:
        pltpu.make_async_copy(rows_a.at[pl.ds(0, sz)],
                              acc.at[pl.ds(base, sz)], sem_c).wait()
        if with_counts:
            pltpu.make_async_copy(ones_v.at[pl.ds(0, sz)],
                                  cnt.at[pl.ds(base, sz)], sem_c).wait()
    if with_counts:
        def orow(i, _):
            ones_v[i, :] = jnp.ones((CW,), _f32)
            return 0
        lax.fori_loop(0, SUPER, orow, 0)
    pltpu.sync_copy(src_all.at[rel, sid], src_v)
    pltpu.sync_copy(dst_all.at[rel, sid], dst_v)
    plsc.subcore_barrier()

    def idx(v, t):
        return v.at[t]

    def gather(t, buf, sem):
        pltpu.async_copy(ycat.at[idx(src_v, t)], buf, sem)

    def gwait(t, buf, sem):
        pltpu.make_async_copy(ycat.at[idx(src_v, t)], buf, sem).wait()

    def scatter(t, buf):
        pltpu.sync_copy(buf, acc.at[idx(dst_v, t)], add=True)
        if with_counts:
            pltpu.async_copy(ones_v, cnt.at[idx(dst_v, t)], sem_c, add=True)

    gather(0, rows_a, sem_a)

    def pair(q, _):
        t = 2 * q
        gwait(t, rows_a, sem_a)
        gather(t + 1, rows_b, sem_b)
        scatter(t, rows_a)
        gwait(t + 1, rows_b, sem_b)
        gather(t + 2, rows_a, sem_a)
        scatter(t + 1, rows_b)
        return 0
    lax.fori_loop(0, SCH // 2 - 1, pair, 0)
    tl = SCH - 2
    gwait(tl, rows_a, sem_a)
    gather(tl + 1, rows_b, sem_b)
    scatter(tl, rows_a)
    gwait(tl + 1, rows_b, sem_b)
    scatter(tl + 1, rows_b)
    if with_counts:
        def cdrain(t, _):
            pltpu.make_async_copy(ones_v, cnt.at[idx(dst_v, t)], sem_c).wait()
            return 0
        lax.fori_loop(0, SCH, cdrain, 0)

    plsc.subcore_barrier()
    pltpu.sync_copy(acc.at[pl.ds(base, ROWS_PER_TILE)],
                    agg_out.at[rel, pl.ds(base, ROWS_PER_TILE)])
    if with_counts:
        pltpu.sync_copy(cnt.at[pl.ds(base, ROWS_PER_TILE)],
                        cnt_out.at[rel, pl.ds(base, ROWS_PER_TILE)])


@functools.cache
def _make_sc(with_counts):
    out_type = [jax.ShapeDtypeStruct((2, NP, H), _bf16)]
    if with_counts:
        out_type.append(jax.ShapeDtypeStruct((2, NP, CW), _f32))
    scratch = [
        pltpu.VMEM((SCH, SUPER), jnp.int32),
        pltpu.VMEM((SCH, SUPER), jnp.int32),
        pltpu.VMEM((SUPER, H), _bf16),
        pltpu.VMEM((SUPER, H), _bf16),
        pltpu.VMEM((SUPER, CW), _f32),
        pltpu.VMEM_SHARED((NP, H), _bf16),
        pltpu.VMEM_SHARED((NP, CW), _f32),
        pltpu.SemaphoreType.DMA,
        pltpu.SemaphoreType.DMA,
        pltpu.SemaphoreType.DMA,
    ]

    if with_counts:
        def body(ycat, src_all, dst_all, agg_out, cnt_out, *s):
            _sc_body(True, ycat, src_all, dst_all, agg_out, cnt_out, *s)
    else:
        def body(ycat, src_all, dst_all, agg_out, *s):
            _sc_body(False, ycat, src_all, dst_all, agg_out, None, *s)

    return pl.kernel(body, mesh=_get_mesh(), out_type=out_type,
                     scratch_types=scratch,
                     compiler_params=pltpu.CompilerParams(
                         use_tc_tiling_on_sc=False))


_BLK = 512
_GRID = NP // _BLK
_GRID2 = 2 * _GRID


def _k1_body(x_ref, wl_ref, wr0_ref, wr1_ref, b0_ref, b1_ref,
             ycat_ref, dense_ref):
    x = x_ref[...]
    ycat_ref[...] = jnp.dot(x, wl_ref[0].T, precision=_HIGH).astype(_bf16)
    dense_ref[...] = (jnp.dot(x, wr0_ref[...].T) + b0_ref[...]
                      + jnp.dot(x, wr1_ref[...].T) + b1_ref[...])


def _make_k1(din):
    row = pl.BlockSpec((_BLK, din), lambda i: (i % _GRID, 0))
    hblk = pl.BlockSpec((_BLK, H), lambda i: (i % _GRID, 0))
    return pl.pallas_call(
        _k1_body,
        grid=(_GRID2,),
        in_specs=[row,
                  pl.BlockSpec((1, H, din), lambda i: (i // _GRID, 0, 0)),
                  pl.BlockSpec((H, din), lambda i: (0, 0)),
                  pl.BlockSpec((H, din), lambda i: (0, 0)),
                  pl.BlockSpec((1, H), lambda i: (0, 0)),
                  pl.BlockSpec((1, H), lambda i: (0, 0))],
        out_specs=[pl.BlockSpec((_BLK, H), lambda i: (i, 0)),
                   hblk],
        out_shape=[jax.ShapeDtypeStruct((2 * NP, H), _bf16),
                   jax.ShapeDtypeStruct((NP, H), _f32)],
    )


def _mean(agg0, agg1, c0, c1):
    return (agg0.astype(_f32) / jnp.maximum(c0, 1.0)
            + agg1.astype(_f32) / jnp.maximum(c1, 1.0))


def _k2_body(dense_ref, a0_ref, a1_ref, c0_ref, c1_ref,
             wl_ref, wr0_ref, wr1_ref, b0_ref, b1_ref,
             ycat_ref, dense1_ref):
    m = _mean(a0_ref[0], a1_ref[0], c0_ref[0], c1_ref[0])
    h = jax.nn.relu(dense_ref[...] + m)
    ycat_ref[...] = jnp.dot(h, wl_ref[0].T, precision=_HIGH).astype(_bf16)
    dense1_ref[...] = (jnp.dot(h, wr0_ref[...].T) + b0_ref[...]
                       + jnp.dot(h, wr1_ref[...].T) + b1_ref[...])


_agg_spec0 = pl.BlockSpec((1, _BLK, H), lambda i: (0, i % _GRID, 0))
_agg_spec1 = pl.BlockSpec((1, _BLK, H), lambda i: (1, i % _GRID, 0))
_cnt_spec0 = pl.BlockSpec((1, _BLK, 1), lambda i: (0, i % _GRID, 0))
_cnt_spec1 = pl.BlockSpec((1, _BLK, 1), lambda i: (1, i % _GRID, 0))

_k2 = pl.pallas_call(
    _k2_body,
    grid=(_GRID2,),
    in_specs=[pl.BlockSpec((_BLK, H), lambda i: (i % _GRID, 0)),
              _agg_spec0, _agg_spec1, _cnt_spec0, _cnt_spec1,
              pl.BlockSpec((1, H, H), lambda i: (i // _GRID, 0, 0)),
              pl.BlockSpec((H, H), lambda i: (0, 0)),
              pl.BlockSpec((H, H), lambda i: (0, 0)),
              pl.BlockSpec((1, H), lambda i: (0, 0)),
              pl.BlockSpec((1, H), lambda i: (0, 0))],
    out_specs=[pl.BlockSpec((_BLK, H), lambda i: (i, 0)),
               pl.BlockSpec((_BLK, H), lambda i: (i % _GRID, 0))],
    out_shape=[jax.ShapeDtypeStruct((2 * NP, H), _bf16),
               jax.ShapeDtypeStruct((NP, H), _f32)],
)


def _k3_body(dense_ref, a0_ref, a1_ref, c0_ref, c1_ref,
             wcls_ref, bcls_ref, out_ref):
    m = _mean(a0_ref[0], a1_ref[0], c0_ref[0], c1_ref[0])
    h2 = dense_ref[...] + m
    out_ref[...] = jnp.dot(h2, wcls_ref[...].T) + bcls_ref[...]


_k3 = pl.pallas_call(
    _k3_body,
    grid=(_GRID,),
    in_specs=[pl.BlockSpec((_BLK, H), lambda i: (i, 0)),
              pl.BlockSpec((1, _BLK, H), lambda i: (0, i, 0)),
              pl.BlockSpec((1, _BLK, H), lambda i: (1, i, 0)),
              pl.BlockSpec((1, _BLK, 1), lambda i: (0, i, 0)),
              pl.BlockSpec((1, _BLK, 1), lambda i: (1, i, 0)),
              pl.BlockSpec((C, H), lambda i: (0, 0)),
              pl.BlockSpec((1, C), lambda i: (0, 0))],
    out_specs=pl.BlockSpec((_BLK, C), lambda i: (i, 0)),
    out_shape=jax.ShapeDtypeStruct((NP, C), _f32),
)


def _prep_edges(ei, src_off):
    pad = EPAD - E
    src = jnp.concatenate([ei[0] + src_off,
                           jnp.full((pad,), src_off, jnp.int32)])
    dst = jnp.concatenate([ei[1], jnp.full((pad,), N, jnp.int32)])
    return (src.reshape(16, SCH, SUPER),
            dst.reshape(16, SCH, SUPER))


def kernel(x, edge_index_rel0, edge_index_rel1,
           Wl_0_0, bl_0_0, Wr_0_0, Wl_0_1, bl_0_1, Wr_0_1,
           Wl_1_0, bl_1_0, Wr_1_0, Wl_1_1, bl_1_1, Wr_1_1,
           W_cls, b_cls):
    s0, d0 = _prep_edges(edge_index_rel0, 0)
    s1, d1 = _prep_edges(edge_index_rel1, NP)
    src_all = jnp.stack([s0, s1])
    dst_all = jnp.stack([d0, d1])

    x_p = jnp.pad(x, ((0, NP - N), (0, 0)))

    wl0 = jnp.stack([Wl_0_0, Wl_0_1])
    ycat0, dense0 = _make_k1(D_IN)(x_p, wl0, Wr_0_0, Wr_0_1,
                                   bl_0_0.reshape(1, H), bl_0_1.reshape(1, H))

    agg0, cnt = _make_sc(True)(ycat0, src_all, dst_all)
    cnts = cnt[:, :, 0:1]

    wl1 = jnp.stack([Wl_1_0, Wl_1_1])
    ycat1, dense1 = _k2(dense0, agg0, agg0, cnts, cnts,
                        wl1, Wr_1_0, Wr_1_1,
                        bl_1_0.reshape(1, H), bl_1_1.reshape(1, H))

    agg1 = _make_sc(False)(ycat1, src_all, dst_all)
    if isinstance(agg1, (list, tuple)):
        agg1 = agg1[0]

    out = _k3(dense1, agg1, agg1, cnts, cnts, W_cls, b_cls.reshape(1, C))
    return out[:N]


# speedup vs baseline: 1.7880x; 1.7880x over previous
import functools

import jax
import jax.numpy as jnp
from jax import lax
from jax.experimental import pallas as pl
from jax.experimental.pallas import tpu as pltpu
from jax.experimental.pallas import tpu_sc as plsc

N = 10000
D_IN = 128
H = 64
C = 2
E = 320000

NP = 10240
ROWS_PER_TILE = NP // 16
SUPER = 512
EROWS = E // SUPER
QMAIN = EROWS // 16
SCH = QMAIN + 1
CW = 16

_f32 = jnp.float32
_bf16 = jnp.bfloat16
_HIGH = jax.lax.Precision.HIGHEST


@functools.cache
def _get_mesh():
    return plsc.VectorSubcoreMesh(core_axis_name="c", subcore_axis_name="s")


def _sc_body(with_counts, ycat, e0, e1, agg_out, cnt_out,
             src_v, dst_v, rows_a, rows_b, ones_v, acc, cnt,
             sem_a, sem_b, sem_c):
    rel = lax.axis_index("c")
    sid = lax.axis_index("s")
    base = sid * ROWS_PER_TILE

    def zrow(i, _):
        for c in range(H // 32):
            rows_a[i, pl.ds(c * 32, 32)] = jnp.zeros((32,), _bf16)
        return 0
    lax.fori_loop(0, SUPER, zrow, 0)
    zparts = [(0, SUPER), (SUPER, ROWS_PER_TILE - SUPER)]
    for off, sz in zparts:
        pltpu.async_copy(rows_a.at[pl.ds(0, sz)],
                         acc.at[pl.ds(base + off, sz)], sem_c)
    if with_counts:
        def crow(i, _):
            ones_v[i, :] = jnp.zeros((CW,), _f32)
            return 0
        lax.fori_loop(0, SUPER, crow, 0)
        for off, sz in zparts:
            pltpu.async_copy(ones_v.at[pl.ds(0, sz)],
                             cnt.at[pl.ds(base + off, sz)], sem_c)
    for off, sz in zparts:
        pltpu.make_async_copy(rows_a.at[pl.ds(0, sz)],
                              acc.at[pl.ds(base, sz)], sem_c).wait()
        if with_counts:
            pltpu.make_async_copy(ones_v.at[pl.ds(0, sz)],
                                  cnt.at[pl.ds(base, sz)], sem_c).wait()
    if with_counts:
        def orow(i, _):
            ones_v[i, :] = jnp.ones((CW,), _f32)
            return 0
        lax.fori_loop(0, SUPER, orow, 0)

    @pl.when(rel == 0)
    def _():
        pltpu.sync_copy(e0.at[0, pl.ds(QMAIN * sid, QMAIN)],
                        src_v.at[pl.ds(0, QMAIN)])
        pltpu.sync_copy(e0.at[1, pl.ds(QMAIN * sid, QMAIN)],
                        dst_v.at[pl.ds(0, QMAIN)])
        pltpu.sync_copy(e0.at[0, pl.ds(EROWS - 1, 1)],
                        src_v.at[pl.ds(QMAIN, 1)])
        pltpu.sync_copy(e0.at[1, pl.ds(EROWS - 1, 1)],
                        dst_v.at[pl.ds(QMAIN, 1)])

    @pl.when(rel == 1)
    def _():
        pltpu.sync_copy(e1.at[0, pl.ds(QMAIN * sid, QMAIN)],
                        src_v.at[pl.ds(0, QMAIN)])
        pltpu.sync_copy(e1.at[1, pl.ds(QMAIN * sid, QMAIN)],
                        dst_v.at[pl.ds(0, QMAIN)])
        pltpu.sync_copy(e1.at[0, pl.ds(EROWS - 1, 1)],
                        src_v.at[pl.ds(QMAIN, 1)])
        pltpu.sync_copy(e1.at[1, pl.ds(EROWS - 1, 1)],
                        dst_v.at[pl.ds(QMAIN, 1)])
    plsc.subcore_barrier()

    def gather(t, buf, sem):
        pltpu.async_copy(ycat.at[rel].at[src_v.at[t]], buf, sem)

    def gwait(t, buf, sem):
        pltpu.make_async_copy(ycat.at[rel].at[src_v.at[t]], buf, sem).wait()

    def scatter(t, buf):
        pltpu.sync_copy(buf, acc.at[dst_v.at[t]], add=True)
        if with_counts:
            pltpu.async_copy(ones_v, cnt.at[dst_v.at[t]], sem_c, add=True)

    gather(0, rows_a, sem_a)

    def pair(q, _):
        t = 2 * q
        gwait(t, rows_a, sem_a)
        gather(t + 1, rows_b, sem_b)
        scatter(t, rows_a)
        gwait(t + 1, rows_b, sem_b)
        gather(t + 2, rows_a, sem_a)
        scatter(t + 1, rows_b)
        return 0
    lax.fori_loop(0, (QMAIN - 1) // 2, pair, 0)
    tl = QMAIN - 1
    gwait(tl, rows_a, sem_a)
    scatter(tl, rows_a)

    @pl.when(sid == 0)
    def _():
        gather(QMAIN, rows_b, sem_b)
        gwait(QMAIN, rows_b, sem_b)
        scatter(QMAIN, rows_b)

    if with_counts:
        def cdrain(t, _):
            pltpu.make_async_copy(ones_v, cnt.at[dst_v.at[t]], sem_c).wait()
            return 0
        lax.fori_loop(0, QMAIN, cdrain, 0)

        @pl.when(sid == 0)
        def _():
            pltpu.make_async_copy(ones_v, cnt.at[dst_v.at[QMAIN]],
                                  sem_c).wait()

    plsc.subcore_barrier()
    pltpu.sync_copy(acc.at[pl.ds(base, ROWS_PER_TILE)],
                    agg_out.at[rel, pl.ds(base, ROWS_PER_TILE)])
    if with_counts:
        pltpu.sync_copy(cnt.at[pl.ds(base, ROWS_PER_TILE)],
                        cnt_out.at[rel, pl.ds(base, ROWS_PER_TILE)])


@functools.cache
def _make_sc(with_counts):
    out_type = [jax.ShapeDtypeStruct((2, NP, H), _bf16)]
    if with_counts:
        out_type.append(jax.ShapeDtypeStruct((2, NP, CW), _f32))
    scratch = [
        pltpu.VMEM((SCH, SUPER), jnp.int32),
        pltpu.VMEM((SCH, SUPER), jnp.int32),
        pltpu.VMEM((SUPER, H), _bf16),
        pltpu.VMEM((SUPER, H), _bf16),
        pltpu.VMEM((SUPER, CW), _f32),
        pltpu.VMEM_SHARED((NP, H), _bf16),
        pltpu.VMEM_SHARED((NP, CW), _f32),
        pltpu.SemaphoreType.DMA,
        pltpu.SemaphoreType.DMA,
        pltpu.SemaphoreType.DMA,
    ]

    if with_counts:
        def body(ycat, e0, e1, agg_out, cnt_out, *s):
            _sc_body(True, ycat, e0, e1, agg_out, cnt_out, *s)
    else:
        def body(ycat, e0, e1, agg_out, *s):
            _sc_body(False, ycat, e0, e1, agg_out, None, *s)

    return pl.kernel(body, mesh=_get_mesh(), out_type=out_type,
                     scratch_types=scratch,
                     compiler_params=pltpu.CompilerParams(
                         use_tc_tiling_on_sc=False))


_BLK = 512
_GRID = NP // _BLK
_GRID2 = 2 * _GRID


def _k1_body(x_ref, wl_ref, wr0_ref, wr1_ref, b0_ref, b1_ref,
             ycat_ref, dense_ref):
    x = x_ref[...]
    ycat_ref[0] = jnp.dot(x, wl_ref[0].T, precision=_HIGH).astype(_bf16)
    dense_ref[...] = (jnp.dot(x, wr0_ref[...].T) + b0_ref[...]
                      + jnp.dot(x, wr1_ref[...].T) + b1_ref[...])


def _make_k1(din):
    row = pl.BlockSpec((_BLK, din), lambda i: (i % _GRID, 0))
    hblk = pl.BlockSpec((_BLK, H), lambda i: (i % _GRID, 0))
    return pl.pallas_call(
        _k1_body,
        grid=(_GRID2,),
        in_specs=[row,
                  pl.BlockSpec((1, H, din), lambda i: (i // _GRID, 0, 0)),
                  pl.BlockSpec((H, din), lambda i: (0, 0)),
                  pl.BlockSpec((H, din), lambda i: (0, 0)),
                  pl.BlockSpec((1, H), lambda i: (0, 0)),
                  pl.BlockSpec((1, H), lambda i: (0, 0))],
        out_specs=[pl.BlockSpec((1, _BLK, H),
                                lambda i: (i // _GRID, i % _GRID, 0)),
                   hblk],
        out_shape=[jax.ShapeDtypeStruct((2, NP, H), _bf16),
                   jax.ShapeDtypeStruct((NP, H), _f32)],
    )


def _mean(a0_ref, a1_ref, c0_ref, c1_ref):
    c0 = c0_ref[0][:, 0:1]
    c1 = c1_ref[0][:, 0:1]
    return (a0_ref[0].astype(_f32) / jnp.maximum(c0, 1.0)
            + a1_ref[0].astype(_f32) / jnp.maximum(c1, 1.0))


def _k2_body(dense_ref, a0_ref, a1_ref, c0_ref, c1_ref,
             wl_ref, wr0_ref, wr1_ref, b0_ref, b1_ref,
             ycat_ref, dense1_ref):
    m = _mean(a0_ref, a1_ref, c0_ref, c1_ref)
    h = jax.nn.relu(dense_ref[...] + m)
    ycat_ref[0] = jnp.dot(h, wl_ref[0].T, precision=_HIGH).astype(_bf16)
    dense1_ref[...] = (jnp.dot(h, wr0_ref[...].T) + b0_ref[...]
                       + jnp.dot(h, wr1_ref[...].T) + b1_ref[...])


_agg_spec0 = pl.BlockSpec((1, _BLK, H), lambda i: (0, i % _GRID, 0))
_agg_spec1 = pl.BlockSpec((1, _BLK, H), lambda i: (1, i % _GRID, 0))
_cnt_spec0 = pl.BlockSpec((1, _BLK, CW), lambda i: (0, i % _GRID, 0))
_cnt_spec1 = pl.BlockSpec((1, _BLK, CW), lambda i: (1, i % _GRID, 0))

_k2 = pl.pallas_call(
    _k2_body,
    grid=(_GRID2,),
    in_specs=[pl.BlockSpec((_BLK, H), lambda i: (i % _GRID, 0)),
              _agg_spec0, _agg_spec1, _cnt_spec0, _cnt_spec1,
              pl.BlockSpec((1, H, H), lambda i: (i // _GRID, 0, 0)),
              pl.BlockSpec((H, H), lambda i: (0, 0)),
              pl.BlockSpec((H, H), lambda i: (0, 0)),
              pl.BlockSpec((1, H), lambda i: (0, 0)),
              pl.BlockSpec((1, H), lambda i: (0, 0))],
    out_specs=[pl.BlockSpec((1, _BLK, H),
                            lambda i: (i // _GRID, i % _GRID, 0)),
               pl.BlockSpec((_BLK, H), lambda i: (i % _GRID, 0))],
    out_shape=[jax.ShapeDtypeStruct((2, NP, H), _bf16),
               jax.ShapeDtypeStruct((NP, H), _f32)],
)


def _k3_body(dense_ref, a0_ref, a1_ref, c0_ref, c1_ref,
             wcls_ref, bcls_ref, out_ref):
    m = _mean(a0_ref, a1_ref, c0_ref, c1_ref)
    h2 = dense_ref[...] + m
    out_ref[...] = jnp.dot(h2, wcls_ref[...].T) + bcls_ref[...]


_k3 = pl.pallas_call(
    _k3_body,
    grid=(_GRID,),
    in_specs=[pl.BlockSpec((_BLK, H), lambda i: (i, 0)),
              pl.BlockSpec((1, _BLK, H), lambda i: (0, i, 0)),
              pl.BlockSpec((1, _BLK, H), lambda i: (1, i, 0)),
              pl.BlockSpec((1, _BLK, CW), lambda i: (0, i, 0)),
              pl.BlockSpec((1, _BLK, CW), lambda i: (1, i, 0)),
              pl.BlockSpec((C, H), lambda i: (0, 0)),
              pl.BlockSpec((1, C), lambda i: (0, 0))],
    out_specs=pl.BlockSpec((_BLK, C), lambda i: (i, 0)),
    out_shape=jax.ShapeDtypeStruct((NP, C), _f32),
)


def kernel(x, edge_index_rel0, edge_index_rel1,
           Wl_0_0, bl_0_0, Wr_0_0, Wl_0_1, bl_0_1, Wr_0_1,
           Wl_1_0, bl_1_0, Wr_1_0, Wl_1_1, bl_1_1, Wr_1_1,
           W_cls, b_cls):
    e0 = edge_index_rel0.reshape(2, EROWS, SUPER)
    e1 = edge_index_rel1.reshape(2, EROWS, SUPER)

    wl0 = jnp.stack([Wl_0_0, Wl_0_1])
    ycat0, dense0 = _make_k1(D_IN)(x, wl0, Wr_0_0, Wr_0_1,
                                   bl_0_0.reshape(1, H), bl_0_1.reshape(1, H))

    agg0, cnt = _make_sc(True)(ycat0, e0, e1)

    wl1 = jnp.stack([Wl_1_0, Wl_1_1])
    ycat1, dense1 = _k2(dense0, agg0, agg0, cnt, cnt,
                        wl1, Wr_1_0, Wr_1_1,
                        bl_1_0.reshape(1, H), bl_1_1.reshape(1, H))

    agg1 = _make_sc(False)(ycat1, e0, e1)
    if isinstance(agg1, (list, tuple)):
        agg1 = agg1[0]

    out = _k3(dense1, agg1, agg1, cnt, cnt, W_cls, b_cls.reshape(1, C))
    return out[:N]


# speedup vs baseline: 1.7922x; 1.0024x over previous
import functools

import jax
import jax.numpy as jnp
from jax import lax
from jax.experimental import pallas as pl
from jax.experimental.pallas import tpu as pltpu
from jax.experimental.pallas import tpu_sc as plsc

N = 10000
D_IN = 128
H = 64
C = 2
E = 320000

NP = 10240
ROWS_PER_TILE = NP // 16
SUPER = 512
EROWS = E // SUPER
QMAIN = EROWS // 16
SCH = QMAIN + 1
CW = 16

_f32 = jnp.float32
_bf16 = jnp.bfloat16
_HIGH = jax.lax.Precision.HIGHEST


@functools.cache
def _get_mesh():
    return plsc.VectorSubcoreMesh(core_axis_name="c", subcore_axis_name="s")


def _sc_body(with_counts, ycat, e0, e1, agg_out, cnt_out,
             src_v, dst_v, rows_a, rows_b, ones_v, acc, cnt,
             sem_a, sem_b, sem_c):
    rel = lax.axis_index("c")
    sid = lax.axis_index("s")
    base = sid * ROWS_PER_TILE

    def zrow(i, _):
        for c in range(H // 32):
            rows_a[i, pl.ds(c * 32, 32)] = jnp.zeros((32,), _bf16)
        return 0
    lax.fori_loop(0, SUPER, zrow, 0)
    zparts = [(0, SUPER), (SUPER, ROWS_PER_TILE - SUPER)]
    for off, sz in zparts:
        pltpu.async_copy(rows_a.at[pl.ds(0, sz)],
                         acc.at[pl.ds(base + off, sz)], sem_c)
    if with_counts:
        def crow(i, _):
            ones_v[i, :] = jnp.zeros((CW,), _f32)
            return 0
        lax.fori_loop(0, SUPER, crow, 0)
        for off, sz in zparts:
            pltpu.async_copy(ones_v.at[pl.ds(0, sz)],
                             cnt.at[pl.ds(base + off, sz)], sem_c)
    for off, sz in zparts:
        pltpu.make_async_copy(rows_a.at[pl.ds(0, sz)],
                              acc.at[pl.ds(base, sz)], sem_c).wait()
        if with_counts:
            pltpu.make_async_copy(ones_v.at[pl.ds(0, sz)],
                                  cnt.at[pl.ds(base, sz)], sem_c).wait()
    if with_counts:
        def orow(i, _):
            ones_v[i, :] = jnp.ones((CW,), _f32)
            return 0
        lax.fori_loop(0, SUPER, orow, 0)

    @pl.when(rel == 0)
    def _():
        pltpu.sync_copy(e0.at[0, pl.ds(QMAIN * sid, QMAIN)],
                        src_v.at[pl.ds(0, QMAIN)])
        pltpu.sync_copy(e0.at[1, pl.ds(QMAIN * sid, QMAIN)],
                        dst_v.at[pl.ds(0, QMAIN)])
        pltpu.sync_copy(e0.at[0, pl.ds(EROWS - 1, 1)],
                        src_v.at[pl.ds(QMAIN, 1)])
        pltpu.sync_copy(e0.at[1, pl.ds(EROWS - 1, 1)],
                        dst_v.at[pl.ds(QMAIN, 1)])

    @pl.when(rel == 1)
    def _():
        pltpu.sync_copy(e1.at[0, pl.ds(QMAIN * sid, QMAIN)],
                        src_v.at[pl.ds(0, QMAIN)])
        pltpu.sync_copy(e1.at[1, pl.ds(QMAIN * sid, QMAIN)],
                        dst_v.at[pl.ds(0, QMAIN)])
        pltpu.sync_copy(e1.at[0, pl.ds(EROWS - 1, 1)],
                        src_v.at[pl.ds(QMAIN, 1)])
        pltpu.sync_copy(e1.at[1, pl.ds(EROWS - 1, 1)],
                        dst_v.at[pl.ds(QMAIN, 1)])
    plsc.subcore_barrier()

    def gather(t, buf, sem):
        pltpu.async_copy(ycat.at[rel].at[src_v.at[t]], buf, sem)

    def gwait(t, buf, sem):
        pltpu.make_async_copy(ycat.at[rel].at[src_v.at[t]], buf, sem).wait()

    def scatter(t, buf):
        pltpu.sync_copy(buf, acc.at[dst_v.at[t]], add=True)
        if with_counts:
            pltpu.async_copy(ones_v, cnt.at[dst_v.at[t]], sem_c, add=True)

    gather(0, rows_a, sem_a)

    def pair(q, _):
        t = 2 * q
        gwait(t, rows_a, sem_a)
        gather(t + 1, rows_b, sem_b)
        scatter(t, rows_a)
        gwait(t + 1, rows_b, sem_b)
        gather(t + 2, rows_a, sem_a)
        scatter(t + 1, rows_b)
        return 0
    lax.fori_loop(0, (QMAIN - 1) // 2, pair, 0)
    tl = QMAIN - 1
    gwait(tl, rows_a, sem_a)
    scatter(tl, rows_a)

    @pl.when(sid == 0)
    def _():
        gather(QMAIN, rows_b, sem_b)
        gwait(QMAIN, rows_b, sem_b)
        scatter(QMAIN, rows_b)

    if with_counts:
        def cdrain(t, _):
            pltpu.make_async_copy(ones_v, cnt.at[dst_v.at[t]], sem_c).wait()
            return 0
        lax.fori_loop(0, QMAIN, cdrain, 0)

        @pl.when(sid == 0)
        def _():
            pltpu.make_async_copy(ones_v, cnt.at[dst_v.at[QMAIN]],
                                  sem_c).wait()

    plsc.subcore_barrier()
    pltpu.sync_copy(acc.at[pl.ds(base, ROWS_PER_TILE)],
                    agg_out.at[rel, pl.ds(base, ROWS_PER_TILE)])
    if with_counts:
        pltpu.sync_copy(cnt.at[pl.ds(base, ROWS_PER_TILE)],
                        cnt_out.at[rel, pl.ds(base, ROWS_PER_TILE)])


@functools.cache
def _make_sc(with_counts):
    out_type = [jax.ShapeDtypeStruct((2, NP, H), _bf16)]
    if with_counts:
        out_type.append(jax.ShapeDtypeStruct((2, NP, CW), _f32))
    scratch = [
        pltpu.VMEM((SCH, SUPER), jnp.int32),
        pltpu.VMEM((SCH, SUPER), jnp.int32),
        pltpu.VMEM((SUPER, H), _bf16),
        pltpu.VMEM((SUPER, H), _bf16),
        pltpu.VMEM((SUPER, CW), _f32),
        pltpu.VMEM_SHARED((NP, H), _bf16),
        pltpu.VMEM_SHARED((NP, CW), _f32),
        pltpu.SemaphoreType.DMA,
        pltpu.SemaphoreType.DMA,
        pltpu.SemaphoreType.DMA,
    ]

    if with_counts:
        def body(ycat, e0, e1, agg_out, cnt_out, *s):
            _sc_body(True, ycat, e0, e1, agg_out, cnt_out, *s)
    else:
        def body(ycat, e0, e1, agg_out, *s):
            _sc_body(False, ycat, e0, e1, agg_out, None, *s)

    return pl.kernel(body, mesh=_get_mesh(), out_type=out_type,
                     scratch_types=scratch,
                     compiler_params=pltpu.CompilerParams(
                         use_tc_tiling_on_sc=False,
                         skip_device_barrier=True))


_BLK = 512
_GRID = NP // _BLK
_GRID2 = 2 * _GRID


def _k1_body(x_ref, wl_ref, wr0_ref, wr1_ref, b0_ref, b1_ref,
             ycat_ref, dense_ref):
    x = x_ref[...]
    ycat_ref[0] = jnp.dot(x, wl_ref[0].T, precision=_HIGH).astype(_bf16)
    dense_ref[...] = (jnp.dot(x, wr0_ref[...].T) + b0_ref[...]
                      + jnp.dot(x, wr1_ref[...].T) + b1_ref[...])


def _make_k1(din):
    row = pl.BlockSpec((_BLK, din), lambda i: (i % _GRID, 0))
    hblk = pl.BlockSpec((_BLK, H), lambda i: (i % _GRID, 0))
    return pl.pallas_call(
        _k1_body,
        grid=(_GRID2,),
        in_specs=[row,
                  pl.BlockSpec((1, H, din), lambda i: (i // _GRID, 0, 0)),
                  pl.BlockSpec((H, din), lambda i: (0, 0)),
                  pl.BlockSpec((H, din), lambda i: (0, 0)),
                  pl.BlockSpec((1, H), lambda i: (0, 0)),
                  pl.BlockSpec((1, H), lambda i: (0, 0))],
        out_specs=[pl.BlockSpec((1, _BLK, H),
                                lambda i: (i // _GRID, i % _GRID, 0)),
                   hblk],
        out_shape=[jax.ShapeDtypeStruct((2, NP, H), _bf16),
                   jax.ShapeDtypeStruct((NP, H), _f32)],
    )


def _mean(a0_ref, a1_ref, c0_ref, c1_ref):
    c0 = c0_ref[0][:, 0:1]
    c1 = c1_ref[0][:, 0:1]
    return (a0_ref[0].astype(_f32) / jnp.maximum(c0, 1.0)
            + a1_ref[0].astype(_f32) / jnp.maximum(c1, 1.0))


def _k2_body(dense_ref, a0_ref, a1_ref, c0_ref, c1_ref,
             wl_ref, wr0_ref, wr1_ref, b0_ref, b1_ref,
             ycat_ref, dense1_ref):
    m = _mean(a0_ref, a1_ref, c0_ref, c1_ref)
    h = jax.nn.relu(dense_ref[...] + m)
    ycat_ref[0] = jnp.dot(h, wl_ref[0].T, precision=_HIGH).astype(_bf16)
    dense1_ref[...] = (jnp.dot(h, wr0_ref[...].T) + b0_ref[...]
                       + jnp.dot(h, wr1_ref[...].T) + b1_ref[...])


_agg_spec0 = pl.BlockSpec((1, _BLK, H), lambda i: (0, i % _GRID, 0))
_agg_spec1 = pl.BlockSpec((1, _BLK, H), lambda i: (1, i % _GRID, 0))
_cnt_spec0 = pl.BlockSpec((1, _BLK, CW), lambda i: (0, i % _GRID, 0))
_cnt_spec1 = pl.BlockSpec((1, _BLK, CW), lambda i: (1, i % _GRID, 0))

_k2 = pl.pallas_call(
    _k2_body,
    grid=(_GRID2,),
    in_specs=[pl.BlockSpec((_BLK, H), lambda i: (i % _GRID, 0)),
              _agg_spec0, _agg_spec1, _cnt_spec0, _cnt_spec1,
              pl.BlockSpec((1, H, H), lambda i: (i // _GRID, 0, 0)),
              pl.BlockSpec((H, H), lambda i: (0, 0)),
              pl.BlockSpec((H, H), lambda i: (0, 0)),
              pl.BlockSpec((1, H), lambda i: (0, 0)),
              pl.BlockSpec((1, H), lambda i: (0, 0))],
    out_specs=[pl.BlockSpec((1, _BLK, H),
                            lambda i: (i // _GRID, i % _GRID, 0)),
               pl.BlockSpec((_BLK, H), lambda i: (i % _GRID, 0))],
    out_shape=[jax.ShapeDtypeStruct((2, NP, H), _bf16),
               jax.ShapeDtypeStruct((NP, H), _f32)],
)


def _k3_body(dense_ref, a0_ref, a1_ref, c0_ref, c1_ref,
             wcls_ref, bcls_ref, out_ref):
    m = _mean(a0_ref, a1_ref, c0_ref, c1_ref)
    h2 = dense_ref[...] + m
    out_ref[...] = jnp.dot(h2, wcls_ref[...].T) + bcls_ref[...]


_k3 = pl.pallas_call(
    _k3_body,
    grid=(_GRID,),
    in_specs=[pl.BlockSpec((_BLK, H), lambda i: (i, 0)),
              pl.BlockSpec((1, _BLK, H), lambda i: (0, i, 0)),
              pl.BlockSpec((1, _BLK, H), lambda i: (1, i, 0)),
              pl.BlockSpec((1, _BLK, CW), lambda i: (0, i, 0)),
              pl.BlockSpec((1, _BLK, CW), lambda i: (1, i, 0)),
              pl.BlockSpec((C, H), lambda i: (0, 0)),
              pl.BlockSpec((1, C), lambda i: (0, 0))],
    out_specs=pl.BlockSpec((_BLK, C), lambda i: (i, 0)),
    out_shape=jax.ShapeDtypeStruct((NP, C), _f32),
)


def kernel(x, edge_index_rel0, edge_index_rel1,
           Wl_0_0, bl_0_0, Wr_0_0, Wl_0_1, bl_0_1, Wr_0_1,
           Wl_1_0, bl_1_0, Wr_1_0, Wl_1_1, bl_1_1, Wr_1_1,
           W_cls, b_cls):
    e0 = edge_index_rel0.reshape(2, EROWS, SUPER)
    e1 = edge_index_rel1.reshape(2, EROWS, SUPER)

    wl0 = jnp.stack([Wl_0_0, Wl_0_1])
    ycat0, dense0 = _make_k1(D_IN)(x, wl0, Wr_0_0, Wr_0_1,
                                   bl_0_0.reshape(1, H), bl_0_1.reshape(1, H))

    agg0, cnt = _make_sc(True)(ycat0, e0, e1)

    wl1 = jnp.stack([Wl_1_0, Wl_1_1])
    ycat1, dense1 = _k2(dense0, agg0, agg0, cnt, cnt,
                        wl1, Wr_1_0, Wr_1_1,
                        bl_1_0.reshape(1, H), bl_1_1.reshape(1, H))

    agg1 = _make_sc(False)(ycat1, e0, e1)
    if isinstance(agg1, (list, tuple)):
        agg1 = agg1[0]

    out = _k3(dense1, agg1, agg1, cnt, cnt, W_cls, b_cls.reshape(1, C))
    return out[:N]


# speedup vs baseline: 1.8104x; 1.0102x over previous
import functools

import jax
import jax.numpy as jnp
from jax import lax
from jax.experimental import pallas as pl
from jax.experimental.pallas import tpu as pltpu
from jax.experimental.pallas import tpu_sc as plsc

N = 10000
D_IN = 128
H = 64
C = 2
E = 320000

NP = 10240
ROWS_PER_TILE = NP // 16
SUPER = 512
EROWS = E // SUPER
QMAIN = EROWS // 16
SCH = QMAIN + 1
CW = 16

_f32 = jnp.float32
_bf16 = jnp.bfloat16
_HIGH = jax.lax.Precision.HIGHEST


@functools.cache
def _get_mesh():
    return plsc.VectorSubcoreMesh(core_axis_name="c", subcore_axis_name="s")


def _sc_body(with_counts, ycat, e0, e1, agg_out, cnt_out,
             src_v, dst_v, rows_a, rows_b, ones_v, acc, cnt,
             sem_a, sem_b, sem_c):
    rel = lax.axis_index("c")
    sid = lax.axis_index("s")
    base = sid * ROWS_PER_TILE

    def zrow(i, _):
        for c in range(H // 32):
            rows_a[i, pl.ds(c * 32, 32)] = jnp.zeros((32,), _bf16)
        return 0
    lax.fori_loop(0, SUPER, zrow, 0)
    zparts = [(0, SUPER), (SUPER, ROWS_PER_TILE - SUPER)]
    for off, sz in zparts:
        pltpu.async_copy(rows_a.at[pl.ds(0, sz)],
                         acc.at[pl.ds(base + off, sz)], sem_c)
    if with_counts:
        def crow(i, _):
            ones_v[i, :] = jnp.zeros((CW,), _f32)
            return 0
        lax.fori_loop(0, SUPER, crow, 0)
        for off, sz in zparts:
            pltpu.async_copy(ones_v.at[pl.ds(0, sz)],
                             cnt.at[pl.ds(base + off, sz)], sem_c)
    for off, sz in zparts:
        pltpu.make_async_copy(rows_a.at[pl.ds(0, sz)],
                              acc.at[pl.ds(base, sz)], sem_c).wait()
        if with_counts:
            pltpu.make_async_copy(ones_v.at[pl.ds(0, sz)],
                                  cnt.at[pl.ds(base, sz)], sem_c).wait()
    if with_counts:
        def orow(i, _):
            ones_v[i, :] = jnp.ones((CW,), _f32)
            return 0
        lax.fori_loop(0, SUPER, orow, 0)

    @pl.when(rel == 0)
    def _():
        pltpu.sync_copy(e0.at[0, pl.ds(QMAIN * sid, QMAIN)],
                        src_v.at[pl.ds(0, QMAIN)])
        pltpu.sync_copy(e0.at[1, pl.ds(QMAIN * sid, QMAIN)],
                        dst_v.at[pl.ds(0, QMAIN)])
        pltpu.sync_copy(e0.at[0, pl.ds(EROWS - 1, 1)],
                        src_v.at[pl.ds(QMAIN, 1)])
        pltpu.sync_copy(e0.at[1, pl.ds(EROWS - 1, 1)],
                        dst_v.at[pl.ds(QMAIN, 1)])

    @pl.when(rel == 1)
    def _():
        pltpu.sync_copy(e1.at[0, pl.ds(QMAIN * sid, QMAIN)],
                        src_v.at[pl.ds(0, QMAIN)])
        pltpu.sync_copy(e1.at[1, pl.ds(QMAIN * sid, QMAIN)],
                        dst_v.at[pl.ds(0, QMAIN)])
        pltpu.sync_copy(e1.at[0, pl.ds(EROWS - 1, 1)],
                        src_v.at[pl.ds(QMAIN, 1)])
        pltpu.sync_copy(e1.at[1, pl.ds(EROWS - 1, 1)],
                        dst_v.at[pl.ds(QMAIN, 1)])
    plsc.subcore_barrier()

    def gather(t, buf, sem):
        pltpu.async_copy(ycat.at[rel].at[src_v.at[t]], buf, sem)

    def gwait(t, buf, sem):
        pltpu.make_async_copy(ycat.at[rel].at[src_v.at[t]], buf, sem).wait()

    def scatter(t, buf):
        pltpu.sync_copy(buf, acc.at[dst_v.at[t]], add=True)
        if with_counts:
            pltpu.async_copy(ones_v, cnt.at[dst_v.at[t]], sem_c, add=True)

    gather(0, rows_a, sem_a)

    def pair(q, _):
        t = 2 * q
        gwait(t, rows_a, sem_a)
        gather(t + 1, rows_b, sem_b)
        scatter(t, rows_a)
        gwait(t + 1, rows_b, sem_b)
        gather(t + 2, rows_a, sem_a)
        scatter(t + 1, rows_b)
        return 0
    lax.fori_loop(0, (QMAIN - 1) // 2, pair, 0)
    tl = QMAIN - 1
    gwait(tl, rows_a, sem_a)
    scatter(tl, rows_a)

    @pl.when(sid == 0)
    def _():
        gather(QMAIN, rows_b, sem_b)
        gwait(QMAIN, rows_b, sem_b)
        scatter(QMAIN, rows_b)

    if with_counts:
        def cdrain(t, _):
            pltpu.make_async_copy(ones_v, cnt.at[dst_v.at[t]], sem_c).wait()
            return 0
        lax.fori_loop(0, QMAIN, cdrain, 0)

        @pl.when(sid == 0)
        def _():
            pltpu.make_async_copy(ones_v, cnt.at[dst_v.at[QMAIN]],
                                  sem_c).wait()

    plsc.subcore_barrier()
    pltpu.sync_copy(acc.at[pl.ds(base, ROWS_PER_TILE)],
                    agg_out.at[rel, pl.ds(base, ROWS_PER_TILE)])
    if with_counts:
        pltpu.sync_copy(cnt.at[pl.ds(base, ROWS_PER_TILE)],
                        cnt_out.at[rel, pl.ds(base, ROWS_PER_TILE)])


@functools.cache
def _make_sc(with_counts):
    out_type = [jax.ShapeDtypeStruct((2, NP, H), _bf16)]
    if with_counts:
        out_type.append(jax.ShapeDtypeStruct((2, NP, CW), _f32))
    scratch = [
        pltpu.VMEM((SCH, SUPER), jnp.int32),
        pltpu.VMEM((SCH, SUPER), jnp.int32),
        pltpu.VMEM((SUPER, H), _bf16),
        pltpu.VMEM((SUPER, H), _bf16),
        pltpu.VMEM((SUPER, CW), _f32),
        pltpu.VMEM_SHARED((NP, H), _bf16),
        pltpu.VMEM_SHARED((NP, CW), _f32),
        pltpu.SemaphoreType.DMA,
        pltpu.SemaphoreType.DMA,
        pltpu.SemaphoreType.DMA,
    ]

    if with_counts:
        def body(ycat, e0, e1, agg_out, cnt_out, *s):
            _sc_body(True, ycat, e0, e1, agg_out, cnt_out, *s)
    else:
        def body(ycat, e0, e1, agg_out, *s):
            _sc_body(False, ycat, e0, e1, agg_out, None, *s)

    return pl.kernel(body, mesh=_get_mesh(), out_type=out_type,
                     scratch_types=scratch,
                     compiler_params=pltpu.CompilerParams(
                         use_tc_tiling_on_sc=False))


_BLK = 512
_GRID = NP // _BLK
_GRID2 = 2 * _GRID


def _k1a_body(x_ref, wl_ref, ycat_ref):
    ycat_ref[0] = jnp.dot(x_ref[...], wl_ref[0].T,
                          precision=_HIGH).astype(_bf16)


_k1a = pl.pallas_call(
    _k1a_body,
    grid=(_GRID2,),
    in_specs=[pl.BlockSpec((_BLK, D_IN), lambda i: (i % _GRID, 0)),
              pl.BlockSpec((1, H, D_IN), lambda i: (i // _GRID, 0, 0))],
    out_specs=pl.BlockSpec((1, _BLK, H),
                           lambda i: (i // _GRID, i % _GRID, 0)),
    out_shape=jax.ShapeDtypeStruct((2, NP, H), _bf16),
)


def _k1b_body(x_ref, wr0_ref, wr1_ref, b0_ref, b1_ref, dense_ref):
    x = x_ref[...]
    dense_ref[...] = (jnp.dot(x, wr0_ref[...].T) + b0_ref[...]
                      + jnp.dot(x, wr1_ref[...].T) + b1_ref[...])


_k1b = pl.pallas_call(
    _k1b_body,
    grid=(_GRID,),
    in_specs=[pl.BlockSpec((_BLK, D_IN), lambda i: (i, 0)),
              pl.BlockSpec((H, D_IN), lambda i: (0, 0)),
              pl.BlockSpec((H, D_IN), lambda i: (0, 0)),
              pl.BlockSpec((1, H), lambda i: (0, 0)),
              pl.BlockSpec((1, H), lambda i: (0, 0))],
    out_specs=pl.BlockSpec((_BLK, H), lambda i: (i, 0)),
    out_shape=jax.ShapeDtypeStruct((NP, H), _f32),
)


def _mean(a0_ref, a1_ref, c0_ref, c1_ref):
    c0 = c0_ref[0][:, 0:1]
    c1 = c1_ref[0][:, 0:1]
    return (a0_ref[0].astype(_f32) / jnp.maximum(c0, 1.0)
            + a1_ref[0].astype(_f32) / jnp.maximum(c1, 1.0))


def _k2a_body(dense_ref, a0_ref, a1_ref, c0_ref, c1_ref, wl_ref, ycat_ref):
    m = _mean(a0_ref, a1_ref, c0_ref, c1_ref)
    h = jax.nn.relu(dense_ref[...] + m)
    ycat_ref[0] = jnp.dot(h, wl_ref[0].T, precision=_HIGH).astype(_bf16)


_agg_spec0 = pl.BlockSpec((1, _BLK, H), lambda i: (0, i % _GRID, 0))
_agg_spec1 = pl.BlockSpec((1, _BLK, H), lambda i: (1, i % _GRID, 0))
_cnt_spec0 = pl.BlockSpec((1, _BLK, CW), lambda i: (0, i % _GRID, 0))
_cnt_spec1 = pl.BlockSpec((1, _BLK, CW), lambda i: (1, i % _GRID, 0))

_k2a = pl.pallas_call(
    _k2a_body,
    grid=(_GRID2,),
    in_specs=[pl.BlockSpec((_BLK, H), lambda i: (i % _GRID, 0)),
              _agg_spec0, _agg_spec1, _cnt_spec0, _cnt_spec1,
              pl.BlockSpec((1, H, H), lambda i: (i // _GRID, 0, 0))],
    out_specs=pl.BlockSpec((1, _BLK, H),
                           lambda i: (i // _GRID, i % _GRID, 0)),
    out_shape=jax.ShapeDtypeStruct((2, NP, H), _bf16),
)


def _k2b_body(dense_ref, a0_ref, a1_ref, c0_ref, c1_ref,
              wr0_ref, wr1_ref, b0_ref, b1_ref, dense1_ref):
    m = _mean(a0_ref, a1_ref, c0_ref, c1_ref)
    h = jax.nn.relu(dense_ref[...] + m)
    dense1_ref[...] = (jnp.dot(h, wr0_ref[...].T) + b0_ref[...]
                       + jnp.dot(h, wr1_ref[...].T) + b1_ref[...])


_k2b = pl.pallas_call(
    _k2b_body,
    grid=(_GRID,),
    in_specs=[pl.BlockSpec((_BLK, H), lambda i: (i, 0)),
              pl.BlockSpec((1, _BLK, H), lambda i: (0, i, 0)),
              pl.BlockSpec((1, _BLK, H), lambda i: (1, i, 0)),
              pl.BlockSpec((1, _BLK, CW), lambda i: (0, i, 0)),
              pl.BlockSpec((1, _BLK, CW), lambda i: (1, i, 0)),
              pl.BlockSpec((H, H), lambda i: (0, 0)),
              pl.BlockSpec((H, H), lambda i: (0, 0)),
              pl.BlockSpec((1, H), lambda i: (0, 0)),
              pl.BlockSpec((1, H), lambda i: (0, 0))],
    out_specs=pl.BlockSpec((_BLK, H), lambda i: (i, 0)),
    out_shape=jax.ShapeDtypeStruct((NP, H), _f32),
)


def _k3_body(dense_ref, a0_ref, a1_ref, c0_ref, c1_ref,
             wcls_ref, bcls_ref, out_ref):
    m = _mean(a0_ref, a1_ref, c0_ref, c1_ref)
    h2 = dense_ref[...] + m
    out_ref[...] = jnp.dot(h2, wcls_ref[...].T) + bcls_ref[...]


_k3 = pl.pallas_call(
    _k3_body,
    grid=(_GRID,),
    in_specs=[pl.BlockSpec((_BLK, H), lambda i: (i, 0)),
              pl.BlockSpec((1, _BLK, H), lambda i: (0, i, 0)),
              pl.BlockSpec((1, _BLK, H), lambda i: (1, i, 0)),
              pl.BlockSpec((1, _BLK, CW), lambda i: (0, i, 0)),
              pl.BlockSpec((1, _BLK, CW), lambda i: (1, i, 0)),
              pl.BlockSpec((C, H), lambda i: (0, 0)),
              pl.BlockSpec((1, C), lambda i: (0, 0))],
    out_specs=pl.BlockSpec((_BLK, C), lambda i: (i, 0)),
    out_shape=jax.ShapeDtypeStruct((NP, C), _f32),
)


def kernel(x, edge_index_rel0, edge_index_rel1,
           Wl_0_0, bl_0_0, Wr_0_0, Wl_0_1, bl_0_1, Wr_0_1,
           Wl_1_0, bl_1_0, Wr_1_0, Wl_1_1, bl_1_1, Wr_1_1,
           W_cls, b_cls):
    e0 = edge_index_rel0.reshape(2, EROWS, SUPER)
    e1 = edge_index_rel1.reshape(2, EROWS, SUPER)

    wl0 = jnp.stack([Wl_0_0, Wl_0_1])
    ycat0 = _k1a(x, wl0)

    agg0, cnt = _make_sc(True)(ycat0, e0, e1)
    dense0 = _k1b(x, Wr_0_0, Wr_0_1,
                  bl_0_0.reshape(1, H), bl_0_1.reshape(1, H))

    wl1 = jnp.stack([Wl_1_0, Wl_1_1])
    ycat1 = _k2a(dense0, agg0, agg0, cnt, cnt, wl1)

    agg1 = _make_sc(False)(ycat1, e0, e1)
    if isinstance(agg1, (list, tuple)):
        agg1 = agg1[0]
    dense1 = _k2b(dense0, agg0, agg0, cnt, cnt, Wr_1_0, Wr_1_1,
                  bl_1_0.reshape(1, H), bl_1_1.reshape(1, H))

    out = _k3(dense1, agg1, agg1, cnt, cnt, W_cls, b_cls.reshape(1, C))
    return out[:N]


# speedup vs baseline: 1.9861x; 1.0970x over previous
import functools

import jax
import jax.numpy as jnp
from jax import lax
from jax.experimental import pallas as pl
from jax.experimental.pallas import tpu as pltpu
from jax.experimental.pallas import tpu_sc as plsc

N = 10000
D_IN = 128
H = 64
C = 2
E = 320000

NP = 10240
ROWS_PER_TILE = NP // 16
SUPER = 512
EROWS = E // SUPER
QMAIN = EROWS // 16
SCH = QMAIN + 1
CW = 16

_f32 = jnp.float32
_bf16 = jnp.bfloat16
_HIGH = jax.lax.Precision.HIGHEST


@functools.cache
def _get_mesh():
    return plsc.VectorSubcoreMesh(core_axis_name="c", subcore_axis_name="s")


def _sc_body(with_counts, ycat, e0, e1, agg_out, cnt_out,
             src_v, dst_v, rows_a, rows_b, ones_v, acc, cnt,
             sem_a, sem_b, sem_c):
    rel = lax.axis_index("c")
    sid = lax.axis_index("s")
    base = sid * ROWS_PER_TILE

    def zrow(i, _):
        for c in range(H // 32):
            rows_a[i, pl.ds(c * 32, 32)] = jnp.zeros((32,), _bf16)
        return 0
    lax.fori_loop(0, SUPER, zrow, 0)
    zparts = [(0, SUPER), (SUPER, ROWS_PER_TILE - SUPER)]
    for off, sz in zparts:
        pltpu.async_copy(rows_a.at[pl.ds(0, sz)],
                         acc.at[pl.ds(base + off, sz)], sem_c)
    if with_counts:
        def crow(i, _):
            ones_v[i, :] = jnp.zeros((CW,), _f32)
            return 0
        lax.fori_loop(0, SUPER, crow, 0)
        for off, sz in zparts:
            pltpu.async_copy(ones_v.at[pl.ds(0, sz)],
                             cnt.at[pl.ds(base + off, sz)], sem_c)
    for off, sz in zparts:
        pltpu.make_async_copy(rows_a.at[pl.ds(0, sz)],
                              acc.at[pl.ds(base, sz)], sem_c).wait()
        if with_counts:
            pltpu.make_async_copy(ones_v.at[pl.ds(0, sz)],
                                  cnt.at[pl.ds(base, sz)], sem_c).wait()
    if with_counts:
        def orow(i, _):
            ones_v[i, :] = jnp.ones((CW,), _f32)
            return 0
        lax.fori_loop(0, SUPER, orow, 0)

    @pl.when(rel == 0)
    def _():
        pltpu.sync_copy(e0.at[0, pl.ds(QMAIN * sid, QMAIN)],
                        src_v.at[pl.ds(0, QMAIN)])
        pltpu.sync_copy(e0.at[1, pl.ds(QMAIN * sid, QMAIN)],
                        dst_v.at[pl.ds(0, QMAIN)])
        pltpu.sync_copy(e0.at[0, pl.ds(EROWS - 1, 1)],
                        src_v.at[pl.ds(QMAIN, 1)])
        pltpu.sync_copy(e0.at[1, pl.ds(EROWS - 1, 1)],
                        dst_v.at[pl.ds(QMAIN, 1)])

    @pl.when(rel == 1)
    def _():
        pltpu.sync_copy(e1.at[0, pl.ds(QMAIN * sid, QMAIN)],
                        src_v.at[pl.ds(0, QMAIN)])
        pltpu.sync_copy(e1.at[1, pl.ds(QMAIN * sid, QMAIN)],
                        dst_v.at[pl.ds(0, QMAIN)])
        pltpu.sync_copy(e1.at[0, pl.ds(EROWS - 1, 1)],
                        src_v.at[pl.ds(QMAIN, 1)])
        pltpu.sync_copy(e1.at[1, pl.ds(EROWS - 1, 1)],
                        dst_v.at[pl.ds(QMAIN, 1)])
    plsc.subcore_barrier()

    def gather(t, buf, sem):
        pltpu.async_copy(ycat.at[rel].at[src_v.at[t]], buf, sem)

    def gwait(t, buf, sem):
        pltpu.make_async_copy(ycat.at[rel].at[src_v.at[t]], buf, sem).wait()

    def scatter(t, buf):
        pltpu.sync_copy(buf, acc.at[dst_v.at[t]], add=True)
        if with_counts:
            pltpu.async_copy(ones_v, cnt.at[dst_v.at[t]], sem_c, add=True)

    gather(0, rows_a, sem_a)

    def pair(q, _):
        t = 2 * q
        gwait(t, rows_a, sem_a)
        gather(t + 1, rows_b, sem_b)
        scatter(t, rows_a)
        gwait(t + 1, rows_b, sem_b)
        gather(t + 2, rows_a, sem_a)
        scatter(t + 1, rows_b)
        return 0
    lax.fori_loop(0, (QMAIN - 1) // 2, pair, 0)
    tl = QMAIN - 1
    gwait(tl, rows_a, sem_a)
    scatter(tl, rows_a)

    @pl.when(sid == 0)
    def _():
        gather(QMAIN, rows_b, sem_b)
        gwait(QMAIN, rows_b, sem_b)
        scatter(QMAIN, rows_b)

    if with_counts:
        def cdrain(t, _):
            pltpu.make_async_copy(ones_v, cnt.at[dst_v.at[t]], sem_c).wait()
            return 0
        lax.fori_loop(0, QMAIN, cdrain, 0)

        @pl.when(sid == 0)
        def _():
            pltpu.make_async_copy(ones_v, cnt.at[dst_v.at[QMAIN]],
                                  sem_c).wait()

    plsc.subcore_barrier()
    pltpu.sync_copy(acc.at[pl.ds(base, ROWS_PER_TILE)],
                    agg_out.at[rel, pl.ds(base, ROWS_PER_TILE)])
    if with_counts:
        pltpu.sync_copy(cnt.at[pl.ds(base, ROWS_PER_TILE)],
                        cnt_out.at[rel, pl.ds(base, ROWS_PER_TILE)])


@functools.cache
def _make_sc(with_counts):
    out_type = [jax.ShapeDtypeStruct((2, NP, H), _bf16)]
    if with_counts:
        out_type.append(jax.ShapeDtypeStruct((2, NP, CW), _f32))
    scratch = [
        pltpu.VMEM((SCH, SUPER), jnp.int32),
        pltpu.VMEM((SCH, SUPER), jnp.int32),
        pltpu.VMEM((SUPER, H), _bf16),
        pltpu.VMEM((SUPER, H), _bf16),
        pltpu.VMEM((SUPER, CW), _f32),
        pltpu.VMEM_SHARED((NP, H), _bf16),
        pltpu.VMEM_SHARED((NP, CW), _f32),
        pltpu.SemaphoreType.DMA,
        pltpu.SemaphoreType.DMA,
        pltpu.SemaphoreType.DMA,
    ]

    if with_counts:
        def body(ycat, e0, e1, agg_out, cnt_out, *s):
            _sc_body(True, ycat, e0, e1, agg_out, cnt_out, *s)
    else:
        def body(ycat, e0, e1, agg_out, *s):
            _sc_body(False, ycat, e0, e1, agg_out, None, *s)

    return pl.kernel(body, mesh=_get_mesh(), out_type=out_type,
                     scratch_types=scratch,
                     compiler_params=pltpu.CompilerParams(
                         use_tc_tiling_on_sc=False))


_BLK = 1024
_GRID = NP // _BLK
_GRID2 = 2 * _GRID


def _k1a_body(x_ref, wl_ref, ycat_ref):
    ycat_ref[0] = jnp.dot(x_ref[...], wl_ref[0].T,
                          precision=_HIGH).astype(_bf16)


_k1a = pl.pallas_call(
    _k1a_body,
    grid=(_GRID2,),
    in_specs=[pl.BlockSpec((_BLK, D_IN), lambda i: (i % _GRID, 0)),
              pl.BlockSpec((1, H, D_IN), lambda i: (i // _GRID, 0, 0))],
    out_specs=pl.BlockSpec((1, _BLK, H),
                           lambda i: (i // _GRID, i % _GRID, 0)),
    out_shape=jax.ShapeDtypeStruct((2, NP, H), _bf16),
)


def _k1b_body(x_ref, wr0_ref, wr1_ref, b0_ref, b1_ref, dense_ref):
    x = x_ref[...]
    dense_ref[...] = (jnp.dot(x, wr0_ref[...].T) + b0_ref[...]
                      + jnp.dot(x, wr1_ref[...].T) + b1_ref[...])


_k1b = pl.pallas_call(
    _k1b_body,
    grid=(_GRID,),
    in_specs=[pl.BlockSpec((_BLK, D_IN), lambda i: (i, 0)),
              pl.BlockSpec((H, D_IN), lambda i: (0, 0)),
              pl.BlockSpec((H, D_IN), lambda i: (0, 0)),
              pl.BlockSpec((1, H), lambda i: (0, 0)),
              pl.BlockSpec((1, H), lambda i: (0, 0))],
    out_specs=pl.BlockSpec((_BLK, H), lambda i: (i, 0)),
    out_shape=jax.ShapeDtypeStruct((NP, H), _f32),
)


def _mean(a0_ref, a1_ref, c0_ref, c1_ref):
    c0 = c0_ref[0][:, 0:1]
    c1 = c1_ref[0][:, 0:1]
    return (a0_ref[0].astype(_f32) / jnp.maximum(c0, 1.0)
            + a1_ref[0].astype(_f32) / jnp.maximum(c1, 1.0))


def _k2a_body(dense_ref, a0_ref, a1_ref, c0_ref, c1_ref, wl_ref, ycat_ref):
    m = _mean(a0_ref, a1_ref, c0_ref, c1_ref)
    h = jax.nn.relu(dense_ref[...] + m)
    ycat_ref[0] = jnp.dot(h, wl_ref[0].T, precision=_HIGH).astype(_bf16)


_agg_spec0 = pl.BlockSpec((1, _BLK, H), lambda i: (0, i % _GRID, 0))
_agg_spec1 = pl.BlockSpec((1, _BLK, H), lambda i: (1, i % _GRID, 0))
_cnt_spec0 = pl.BlockSpec((1, _BLK, CW), lambda i: (0, i % _GRID, 0))
_cnt_spec1 = pl.BlockSpec((1, _BLK, CW), lambda i: (1, i % _GRID, 0))

_k2a = pl.pallas_call(
    _k2a_body,
    grid=(_GRID2,),
    in_specs=[pl.BlockSpec((_BLK, H), lambda i: (i % _GRID, 0)),
              _agg_spec0, _agg_spec1, _cnt_spec0, _cnt_spec1,
              pl.BlockSpec((1, H, H), lambda i: (i // _GRID, 0, 0))],
    out_specs=pl.BlockSpec((1, _BLK, H),
                           lambda i: (i // _GRID, i % _GRID, 0)),
    out_shape=jax.ShapeDtypeStruct((2, NP, H), _bf16),
)


def _k2b_body(dense_ref, a0_ref, a1_ref, c0_ref, c1_ref,
              wr0_ref, wr1_ref, b0_ref, b1_ref, dense1_ref):
    m = _mean(a0_ref, a1_ref, c0_ref, c1_ref)
    h = jax.nn.relu(dense_ref[...] + m)
    dense1_ref[...] = (jnp.dot(h, wr0_ref[...].T) + b0_ref[...]
                       + jnp.dot(h, wr1_ref[...].T) + b1_ref[...])


_k2b = pl.pallas_call(
    _k2b_body,
    grid=(_GRID,),
    in_specs=[pl.BlockSpec((_BLK, H), lambda i: (i, 0)),
              pl.BlockSpec((1, _BLK, H), lambda i: (0, i, 0)),
              pl.BlockSpec((1, _BLK, H), lambda i: (1, i, 0)),
              pl.BlockSpec((1, _BLK, CW), lambda i: (0, i, 0)),
              pl.BlockSpec((1, _BLK, CW), lambda i: (1, i, 0)),
              pl.BlockSpec((H, H), lambda i: (0, 0)),
              pl.BlockSpec((H, H), lambda i: (0, 0)),
              pl.BlockSpec((1, H), lambda i: (0, 0)),
              pl.BlockSpec((1, H), lambda i: (0, 0))],
    out_specs=pl.BlockSpec((_BLK, H), lambda i: (i, 0)),
    out_shape=jax.ShapeDtypeStruct((NP, H), _f32),
)


def _k3_body(dense_ref, a0_ref, a1_ref, c0_ref, c1_ref,
             wcls_ref, bcls_ref, out_ref):
    m = _mean(a0_ref, a1_ref, c0_ref, c1_ref)
    h2 = dense_ref[...] + m
    out_ref[...] = jnp.dot(h2, wcls_ref[...].T) + bcls_ref[...]


_k3 = pl.pallas_call(
    _k3_body,
    grid=(_GRID,),
    in_specs=[pl.BlockSpec((_BLK, H), lambda i: (i, 0)),
              pl.BlockSpec((1, _BLK, H), lambda i: (0, i, 0)),
              pl.BlockSpec((1, _BLK, H), lambda i: (1, i, 0)),
              pl.BlockSpec((1, _BLK, CW), lambda i: (0, i, 0)),
              pl.BlockSpec((1, _BLK, CW), lambda i: (1, i, 0)),
              pl.BlockSpec((C, H), lambda i: (0, 0)),
              pl.BlockSpec((1, C), lambda i: (0, 0))],
    out_specs=pl.BlockSpec((_BLK, C), lambda i: (i, 0)),
    out_shape=jax.ShapeDtypeStruct((NP, C), _f32),
)


def kernel(x, edge_index_rel0, edge_index_rel1,
           Wl_0_0, bl_0_0, Wr_0_0, Wl_0_1, bl_0_1, Wr_0_1,
           Wl_1_0, bl_1_0, Wr_1_0, Wl_1_1, bl_1_1, Wr_1_1,
           W_cls, b_cls):
    e0 = edge_index_rel0.reshape(2, EROWS, SUPER)
    e1 = edge_index_rel1.reshape(2, EROWS, SUPER)

    wl0 = jnp.stack([Wl_0_0, Wl_0_1])
    ycat0 = _k1a(x, wl0)

    agg0, cnt = _make_sc(True)(ycat0, e0, e1)
    dense0 = _k1b(x, Wr_0_0, Wr_0_1,
                  bl_0_0.reshape(1, H), bl_0_1.reshape(1, H))

    wl1 = jnp.stack([Wl_1_0, Wl_1_1])
    ycat1 = _k2a(dense0, agg0, agg0, cnt, cnt, wl1)

    agg1 = _make_sc(False)(ycat1, e0, e1)
    if isinstance(agg1, (list, tuple)):
        agg1 = agg1[0]
    dense1 = _k2b(dense0, agg0, agg0, cnt, cnt, Wr_1_0, Wr_1_1,
                  bl_1_0.reshape(1, H), bl_1_1.reshape(1, H))

    out = _k3(dense1, agg1, agg1, cnt, cnt, W_cls, b_cls.reshape(1, C))
    return out[:N]


# speedup vs baseline: 2.0790x; 1.0468x over previous
import functools

import jax
import jax.numpy as jnp
from jax import lax
from jax.experimental import pallas as pl
from jax.experimental.pallas import tpu as pltpu
from jax.experimental.pallas import tpu_sc as plsc

N = 10000
D_IN = 128
H = 64
C = 2
E = 320000

NP = 10240
ROWS_PER_TILE = NP // 16
SUPER = 512
EROWS = E // SUPER
QMAIN = EROWS // 16
SCH = QMAIN + 1
CW = 16

_f32 = jnp.float32
_bf16 = jnp.bfloat16
_HIGH = jax.lax.Precision.HIGHEST


@functools.cache
def _get_mesh():
    return plsc.VectorSubcoreMesh(core_axis_name="c", subcore_axis_name="s")


def _sc_body(with_counts, ycat, e0, e1, agg_out, cnt_out,
             src_v, dst_v, rows_a, rows_b, ones_v, acc, cnt,
             sem_a, sem_b, sem_c):
    rel = lax.axis_index("c")
    sid = lax.axis_index("s")
    base = sid * ROWS_PER_TILE

    def zrow(i, _):
        for c in range(H // 32):
            rows_a[i, pl.ds(c * 32, 32)] = jnp.zeros((32,), _bf16)
        return 0
    lax.fori_loop(0, SUPER, zrow, 0)
    zparts = [(0, SUPER), (SUPER, ROWS_PER_TILE - SUPER)]
    for off, sz in zparts:
        pltpu.async_copy(rows_a.at[pl.ds(0, sz)],
                         acc.at[pl.ds(base + off, sz)], sem_c)
    if with_counts:
        def crow(i, _):
            ones_v[i, :] = jnp.zeros((CW,), _f32)
            return 0
        lax.fori_loop(0, SUPER, crow, 0)
        for off, sz in zparts:
            pltpu.async_copy(ones_v.at[pl.ds(0, sz)],
                             cnt.at[pl.ds(base + off, sz)], sem_c)
    for off, sz in zparts:
        pltpu.make_async_copy(rows_a.at[pl.ds(0, sz)],
                              acc.at[pl.ds(base, sz)], sem_c).wait()
        if with_counts:
            pltpu.make_async_copy(ones_v.at[pl.ds(0, sz)],
                                  cnt.at[pl.ds(base, sz)], sem_c).wait()
    if with_counts:
        def orow(i, _):
            ones_v[i, :] = jnp.ones((CW,), _f32)
            return 0
        lax.fori_loop(0, SUPER, orow, 0)

    @pl.when(rel == 0)
    def _():
        pltpu.sync_copy(e0.at[0, pl.ds(QMAIN * sid, QMAIN)],
                        src_v.at[pl.ds(0, QMAIN)])
        pltpu.sync_copy(e0.at[1, pl.ds(QMAIN * sid, QMAIN)],
                        dst_v.at[pl.ds(0, QMAIN)])
        pltpu.sync_copy(e0.at[0, pl.ds(EROWS - 1, 1)],
                        src_v.at[pl.ds(QMAIN, 1)])
        pltpu.sync_copy(e0.at[1, pl.ds(EROWS - 1, 1)],
                        dst_v.at[pl.ds(QMAIN, 1)])

    @pl.when(rel == 1)
    def _():
        pltpu.sync_copy(e1.at[0, pl.ds(QMAIN * sid, QMAIN)],
                        src_v.at[pl.ds(0, QMAIN)])
        pltpu.sync_copy(e1.at[1, pl.ds(QMAIN * sid, QMAIN)],
                        dst_v.at[pl.ds(0, QMAIN)])
        pltpu.sync_copy(e1.at[0, pl.ds(EROWS - 1, 1)],
                        src_v.at[pl.ds(QMAIN, 1)])
        pltpu.sync_copy(e1.at[1, pl.ds(EROWS - 1, 1)],
                        dst_v.at[pl.ds(QMAIN, 1)])
    plsc.subcore_barrier()

    def gather(t, buf, sem):
        pltpu.async_copy(ycat.at[rel].at[src_v.at[t]], buf, sem)

    def gwait(t, buf, sem):
        pltpu.make_async_copy(ycat.at[rel].at[src_v.at[t]], buf, sem).wait()

    def scatter(t, buf):
        pltpu.sync_copy(buf, acc.at[dst_v.at[t]], add=True)
        if with_counts:
            pltpu.async_copy(ones_v, cnt.at[dst_v.at[t]], sem_c, add=True)

    gather(0, rows_a, sem_a)

    def pair(q, _):
        t = 2 * q
        gwait(t, rows_a, sem_a)
        gather(t + 1, rows_b, sem_b)
        scatter(t, rows_a)
        gwait(t + 1, rows_b, sem_b)
        gather(t + 2, rows_a, sem_a)
        scatter(t + 1, rows_b)
        return 0
    lax.fori_loop(0, (QMAIN - 1) // 2, pair, 0)
    tl = QMAIN - 1
    gwait(tl, rows_a, sem_a)
    scatter(tl, rows_a)

    @pl.when(sid == 0)
    def _():
        gather(QMAIN, rows_b, sem_b)
        gwait(QMAIN, rows_b, sem_b)
        scatter(QMAIN, rows_b)

    if with_counts:
        def cdrain(t, _):
            pltpu.make_async_copy(ones_v, cnt.at[dst_v.at[t]], sem_c).wait()
            return 0
        lax.fori_loop(0, QMAIN, cdrain, 0)

        @pl.when(sid == 0)
        def _():
            pltpu.make_async_copy(ones_v, cnt.at[dst_v.at[QMAIN]],
                                  sem_c).wait()

    plsc.subcore_barrier()
    pltpu.sync_copy(acc.at[pl.ds(base, ROWS_PER_TILE)],
                    agg_out.at[rel, pl.ds(base, ROWS_PER_TILE)])
    if with_counts:
        pltpu.sync_copy(cnt.at[pl.ds(base, ROWS_PER_TILE)],
                        cnt_out.at[rel, pl.ds(base, ROWS_PER_TILE)])


@functools.cache
def _make_sc(with_counts):
    out_type = [jax.ShapeDtypeStruct((2, NP, H), _bf16)]
    if with_counts:
        out_type.append(jax.ShapeDtypeStruct((2, NP, CW), _f32))
    scratch = [
        pltpu.VMEM((SCH, SUPER), jnp.int32),
        pltpu.VMEM((SCH, SUPER), jnp.int32),
        pltpu.VMEM((SUPER, H), _bf16),
        pltpu.VMEM((SUPER, H), _bf16),
        pltpu.VMEM((SUPER, CW), _f32),
        pltpu.VMEM_SHARED((NP, H), _bf16),
        pltpu.VMEM_SHARED((NP, CW), _f32),
        pltpu.SemaphoreType.DMA,
        pltpu.SemaphoreType.DMA,
        pltpu.SemaphoreType.DMA,
    ]

    if with_counts:
        def body(ycat, e0, e1, agg_out, cnt_out, *s):
            _sc_body(True, ycat, e0, e1, agg_out, cnt_out, *s)
    else:
        def body(ycat, e0, e1, agg_out, *s):
            _sc_body(False, ycat, e0, e1, agg_out, None, *s)

    return pl.kernel(body, mesh=_get_mesh(), out_type=out_type,
                     scratch_types=scratch,
                     compiler_params=pltpu.CompilerParams(
                         use_tc_tiling_on_sc=False))


_BLK = 2048
_GRID = NP // _BLK
_GRID2 = 2 * _GRID


def _k1a_body(x_ref, wl_ref, ycat_ref):
    ycat_ref[0] = jnp.dot(x_ref[...], wl_ref[0].T,
                          precision=_HIGH).astype(_bf16)


_k1a = pl.pallas_call(
    _k1a_body,
    grid=(_GRID2,),
    in_specs=[pl.BlockSpec((_BLK, D_IN), lambda i: (i % _GRID, 0)),
              pl.BlockSpec((1, H, D_IN), lambda i: (i // _GRID, 0, 0))],
    out_specs=pl.BlockSpec((1, _BLK, H),
                           lambda i: (i // _GRID, i % _GRID, 0)),
    out_shape=jax.ShapeDtypeStruct((2, NP, H), _bf16),
)


def _k1b_body(x_ref, wr0_ref, wr1_ref, b0_ref, b1_ref, dense_ref):
    x = x_ref[...]
    dense_ref[...] = (jnp.dot(x, wr0_ref[...].T) + b0_ref[...]
                      + jnp.dot(x, wr1_ref[...].T) + b1_ref[...])


_k1b = pl.pallas_call(
    _k1b_body,
    grid=(_GRID,),
    in_specs=[pl.BlockSpec((_BLK, D_IN), lambda i: (i, 0)),
              pl.BlockSpec((H, D_IN), lambda i: (0, 0)),
              pl.BlockSpec((H, D_IN), lambda i: (0, 0)),
              pl.BlockSpec((1, H), lambda i: (0, 0)),
              pl.BlockSpec((1, H), lambda i: (0, 0))],
    out_specs=pl.BlockSpec((_BLK, H), lambda i: (i, 0)),
    out_shape=jax.ShapeDtypeStruct((NP, H), _f32),
)


def _mean(a0_ref, a1_ref, c0_ref, c1_ref):
    c0 = c0_ref[0][:, 0:1]
    c1 = c1_ref[0][:, 0:1]
    return (a0_ref[0].astype(_f32) / jnp.maximum(c0, 1.0)
            + a1_ref[0].astype(_f32) / jnp.maximum(c1, 1.0))


def _k2a_body(dense_ref, a0_ref, a1_ref, c0_ref, c1_ref, wl_ref, ycat_ref):
    m = _mean(a0_ref, a1_ref, c0_ref, c1_ref)
    h = jax.nn.relu(dense_ref[...] + m)
    ycat_ref[0] = jnp.dot(h, wl_ref[0].T, precision=_HIGH).astype(_bf16)


_agg_spec0 = pl.BlockSpec((1, _BLK, H), lambda i: (0, i % _GRID, 0))
_agg_spec1 = pl.BlockSpec((1, _BLK, H), lambda i: (1, i % _GRID, 0))
_cnt_spec0 = pl.BlockSpec((1, _BLK, CW), lambda i: (0, i % _GRID, 0))
_cnt_spec1 = pl.BlockSpec((1, _BLK, CW), lambda i: (1, i % _GRID, 0))

_k2a = pl.pallas_call(
    _k2a_body,
    grid=(_GRID2,),
    in_specs=[pl.BlockSpec((_BLK, H), lambda i: (i % _GRID, 0)),
              _agg_spec0, _agg_spec1, _cnt_spec0, _cnt_spec1,
              pl.BlockSpec((1, H, H), lambda i: (i // _GRID, 0, 0))],
    out_specs=pl.BlockSpec((1, _BLK, H),
                           lambda i: (i // _GRID, i % _GRID, 0)),
    out_shape=jax.ShapeDtypeStruct((2, NP, H), _bf16),
)


def _k2b_body(dense_ref, a0_ref, a1_ref, c0_ref, c1_ref,
              wr0_ref, wr1_ref, b0_ref, b1_ref, dense1_ref):
    m = _mean(a0_ref, a1_ref, c0_ref, c1_ref)
    h = jax.nn.relu(dense_ref[...] + m)
    dense1_ref[...] = (jnp.dot(h, wr0_ref[...].T) + b0_ref[...]
                       + jnp.dot(h, wr1_ref[...].T) + b1_ref[...])


_k2b = pl.pallas_call(
    _k2b_body,
    grid=(_GRID,),
    in_specs=[pl.BlockSpec((_BLK, H), lambda i: (i, 0)),
              pl.BlockSpec((1, _BLK, H), lambda i: (0, i, 0)),
              pl.BlockSpec((1, _BLK, H), lambda i: (1, i, 0)),
              pl.BlockSpec((1, _BLK, CW), lambda i: (0, i, 0)),
              pl.BlockSpec((1, _BLK, CW), lambda i: (1, i, 0)),
              pl.BlockSpec((H, H), lambda i: (0, 0)),
              pl.BlockSpec((H, H), lambda i: (0, 0)),
              pl.BlockSpec((1, H), lambda i: (0, 0)),
              pl.BlockSpec((1, H), lambda i: (0, 0))],
    out_specs=pl.BlockSpec((_BLK, H), lambda i: (i, 0)),
    out_shape=jax.ShapeDtypeStruct((NP, H), _f32),
)


def _k3_body(dense_ref, a0_ref, a1_ref, c0_ref, c1_ref,
             wcls_ref, bcls_ref, out_ref):
    m = _mean(a0_ref, a1_ref, c0_ref, c1_ref)
    h2 = dense_ref[...] + m
    out_ref[...] = jnp.dot(h2, wcls_ref[...].T) + bcls_ref[...]


_k3 = pl.pallas_call(
    _k3_body,
    grid=(_GRID,),
    in_specs=[pl.BlockSpec((_BLK, H), lambda i: (i, 0)),
              pl.BlockSpec((1, _BLK, H), lambda i: (0, i, 0)),
              pl.BlockSpec((1, _BLK, H), lambda i: (1, i, 0)),
              pl.BlockSpec((1, _BLK, CW), lambda i: (0, i, 0)),
              pl.BlockSpec((1, _BLK, CW), lambda i: (1, i, 0)),
              pl.BlockSpec((C, H), lambda i: (0, 0)),
              pl.BlockSpec((1, C), lambda i: (0, 0))],
    out_specs=pl.BlockSpec((_BLK, C), lambda i: (i, 0)),
    out_shape=jax.ShapeDtypeStruct((NP, C), _f32),
)


def kernel(x, edge_index_rel0, edge_index_rel1,
           Wl_0_0, bl_0_0, Wr_0_0, Wl_0_1, bl_0_1, Wr_0_1,
           Wl_1_0, bl_1_0, Wr_1_0, Wl_1_1, bl_1_1, Wr_1_1,
           W_cls, b_cls):
    e0 = edge_index_rel0.reshape(2, EROWS, SUPER)
    e1 = edge_index_rel1.reshape(2, EROWS, SUPER)

    wl0 = jnp.stack([Wl_0_0, Wl_0_1])
    ycat0 = _k1a(x, wl0)

    agg0, cnt = _make_sc(True)(ycat0, e0, e1)
    dense0 = _k1b(x, Wr_0_0, Wr_0_1,
                  bl_0_0.reshape(1, H), bl_0_1.reshape(1, H))

    wl1 = jnp.stack([Wl_1_0, Wl_1_1])
    ycat1 = _k2a(dense0, agg0, agg0, cnt, cnt, wl1)

    agg1 = _make_sc(False)(ycat1, e0, e1)
    if isinstance(agg1, (list, tuple)):
        agg1 = agg1[0]
    dense1 = _k2b(dense0, agg0, agg0, cnt, cnt, Wr_1_0, Wr_1_1,
                  bl_1_0.reshape(1, H), bl_1_1.reshape(1, H))

    out = _k3(dense1, agg1, agg1, cnt, cnt, W_cls, b_cls.reshape(1, C))
    return out[:N]


# speedup vs baseline: 2.1219x; 1.0206x over previous
import functools

import jax
import jax.numpy as jnp
from jax import lax
from jax.experimental import pallas as pl
from jax.experimental.pallas import tpu as pltpu
from jax.experimental.pallas import tpu_sc as plsc

N = 10000
D_IN = 128
H = 64
C = 2
E = 320000

NP = 10240
ROWS_PER_TILE = NP // 16
SUPER = 512
EROWS = E // SUPER
QMAIN = EROWS // 16
SCH = QMAIN + 1
CW = 16

_f32 = jnp.float32
_bf16 = jnp.bfloat16
_HIGH = jax.lax.Precision.HIGHEST


@functools.cache
def _get_mesh():
    return plsc.VectorSubcoreMesh(core_axis_name="c", subcore_axis_name="s")


def _sc_body(with_counts, ycat, e0, e1, agg_out, cnt_out,
             src_v, dst_v, rows_a, rows_b, ones_v, acc, cnt,
             sem_a, sem_b, sem_c):
    rel = lax.axis_index("c")
    sid = lax.axis_index("s")
    base = sid * ROWS_PER_TILE

    def zrow(i, _):
        for c in range(H // 32):
            rows_a[i, pl.ds(c * 32, 32)] = jnp.zeros((32,), _bf16)
        return 0
    lax.fori_loop(0, SUPER, zrow, 0)
    zparts = [(0, SUPER), (SUPER, ROWS_PER_TILE - SUPER)]
    for off, sz in zparts:
        pltpu.async_copy(rows_a.at[pl.ds(0, sz)],
                         acc.at[pl.ds(base + off, sz)], sem_c)
    if with_counts:
        def crow(i, _):
            ones_v[i, :] = jnp.zeros((CW,), _f32)
            return 0
        lax.fori_loop(0, SUPER, crow, 0)
        for off, sz in zparts:
            pltpu.async_copy(ones_v.at[pl.ds(0, sz)],
                             cnt.at[pl.ds(base + off, sz)], sem_c)
    for off, sz in zparts:
        pltpu.make_async_copy(rows_a.at[pl.ds(0, sz)],
                              acc.at[pl.ds(base, sz)], sem_c).wait()
        if with_counts:
            pltpu.make_async_copy(ones_v.at[pl.ds(0, sz)],
                                  cnt.at[pl.ds(base, sz)], sem_c).wait()
    if with_counts:
        def orow(i, _):
            ones_v[i, :] = jnp.ones((CW,), _f32)
            return 0
        lax.fori_loop(0, SUPER, orow, 0)

    @pl.when(rel == 0)
    def _():
        pltpu.sync_copy(e0.at[0, pl.ds(QMAIN * sid, QMAIN)],
                        src_v.at[pl.ds(0, QMAIN)])
        pltpu.sync_copy(e0.at[1, pl.ds(QMAIN * sid, QMAIN)],
                        dst_v.at[pl.ds(0, QMAIN)])
        pltpu.sync_copy(e0.at[0, pl.ds(EROWS - 1, 1)],
                        src_v.at[pl.ds(QMAIN, 1)])
        pltpu.sync_copy(e0.at[1, pl.ds(EROWS - 1, 1)],
                        dst_v.at[pl.ds(QMAIN, 1)])

    @pl.when(rel == 1)
    def _():
        pltpu.sync_copy(e1.at[0, pl.ds(QMAIN * sid, QMAIN)],
                        src_v.at[pl.ds(0, QMAIN)])
        pltpu.sync_copy(e1.at[1, pl.ds(QMAIN * sid, QMAIN)],
                        dst_v.at[pl.ds(0, QMAIN)])
        pltpu.sync_copy(e1.at[0, pl.ds(EROWS - 1, 1)],
                        src_v.at[pl.ds(QMAIN, 1)])
        pltpu.sync_copy(e1.at[1, pl.ds(EROWS - 1, 1)],
                        dst_v.at[pl.ds(QMAIN, 1)])
    plsc.subcore_barrier()

    def gather(t, buf, sem):
        pltpu.async_copy(ycat.at[rel].at[src_v.at[t]], buf, sem)

    def gwait(t, buf, sem):
        pltpu.make_async_copy(ycat.at[rel].at[src_v.at[t]], buf, sem).wait()

    def scatter(t, buf):
        pltpu.sync_copy(buf, acc.at[dst_v.at[t]], add=True)
        if with_counts:
            pltpu.async_copy(ones_v, cnt.at[dst_v.at[t]], sem_c, add=True)

    gather(0, rows_a, sem_a)

    def pair(q, _):
        t = 2 * q
        gwait(t, rows_a, sem_a)
        gather(t + 1, rows_b, sem_b)
        scatter(t, rows_a)
        gwait(t + 1, rows_b, sem_b)
        gather(t + 2, rows_a, sem_a)
        scatter(t + 1, rows_b)
        return 0
    lax.fori_loop(0, (QMAIN - 1) // 2, pair, 0)
    tl = QMAIN - 1
    gwait(tl, rows_a, sem_a)
    scatter(tl, rows_a)

    @pl.when(sid == 0)
    def _():
        gather(QMAIN, rows_b, sem_b)
        gwait(QMAIN, rows_b, sem_b)
        scatter(QMAIN, rows_b)

    if with_counts:
        def cdrain(t, _):
            pltpu.make_async_copy(ones_v, cnt.at[dst_v.at[t]], sem_c).wait()
            return 0
        lax.fori_loop(0, QMAIN, cdrain, 0)

        @pl.when(sid == 0)
        def _():
            pltpu.make_async_copy(ones_v, cnt.at[dst_v.at[QMAIN]],
                                  sem_c).wait()

    plsc.subcore_barrier()
    pltpu.sync_copy(acc.at[pl.ds(base, ROWS_PER_TILE)],
                    agg_out.at[rel, pl.ds(base, ROWS_PER_TILE)])
    if with_counts:
        pltpu.sync_copy(cnt.at[pl.ds(base, ROWS_PER_TILE)],
                        cnt_out.at[rel, pl.ds(base, ROWS_PER_TILE)])


@functools.cache
def _make_sc(with_counts):
    out_type = [jax.ShapeDtypeStruct((2, NP, H), _bf16)]
    if with_counts:
        out_type.append(jax.ShapeDtypeStruct((2, NP, CW), _f32))
    scratch = [
        pltpu.VMEM((SCH, SUPER), jnp.int32),
        pltpu.VMEM((SCH, SUPER), jnp.int32),
        pltpu.VMEM((SUPER, H), _bf16),
        pltpu.VMEM((SUPER, H), _bf16),
        pltpu.VMEM((SUPER, CW), _f32),
        pltpu.VMEM_SHARED((NP, H), _bf16),
        pltpu.VMEM_SHARED((NP, CW), _f32),
        pltpu.SemaphoreType.DMA,
        pltpu.SemaphoreType.DMA,
        pltpu.SemaphoreType.DMA,
    ]

    if with_counts:
        def body(ycat, e0, e1, agg_out, cnt_out, *s):
            _sc_body(True, ycat, e0, e1, agg_out, cnt_out, *s)
    else:
        def body(ycat, e0, e1, agg_out, *s):
            _sc_body(False, ycat, e0, e1, agg_out, None, *s)

    return pl.kernel(body, mesh=_get_mesh(), out_type=out_type,
                     scratch_types=scratch,
                     compiler_params=pltpu.CompilerParams(
                         use_tc_tiling_on_sc=False))


_BLK = 5120
_GRID = NP // _BLK
_GRID2 = 2 * _GRID


def _k1a_body(x_ref, wl_ref, ycat_ref):
    ycat_ref[0] = jnp.dot(x_ref[...], wl_ref[0].T,
                          precision=_HIGH).astype(_bf16)


_k1a = pl.pallas_call(
    _k1a_body,
    grid=(_GRID2,),
    in_specs=[pl.BlockSpec((_BLK, D_IN), lambda i: (i % _GRID, 0)),
              pl.BlockSpec((1, H, D_IN), lambda i: (i // _GRID, 0, 0))],
    out_specs=pl.BlockSpec((1, _BLK, H),
                           lambda i: (i // _GRID, i % _GRID, 0)),
    out_shape=jax.ShapeDtypeStruct((2, NP, H), _bf16),
)


def _k1b_body(x_ref, wr0_ref, wr1_ref, b0_ref, b1_ref, dense_ref):
    x = x_ref[...]
    dense_ref[...] = (jnp.dot(x, wr0_ref[...].T) + b0_ref[...]
                      + jnp.dot(x, wr1_ref[...].T) + b1_ref[...])


_k1b = pl.pallas_call(
    _k1b_body,
    grid=(_GRID,),
    in_specs=[pl.BlockSpec((_BLK, D_IN), lambda i: (i, 0)),
              pl.BlockSpec((H, D_IN), lambda i: (0, 0)),
              pl.BlockSpec((H, D_IN), lambda i: (0, 0)),
              pl.BlockSpec((1, H), lambda i: (0, 0)),
              pl.BlockSpec((1, H), lambda i: (0, 0))],
    out_specs=pl.BlockSpec((_BLK, H), lambda i: (i, 0)),
    out_shape=jax.ShapeDtypeStruct((NP, H), _f32),
)


def _mean(a0_ref, a1_ref, c0_ref, c1_ref):
    c0 = c0_ref[0][:, 0:1]
    c1 = c1_ref[0][:, 0:1]
    return (a0_ref[0].astype(_f32) / jnp.maximum(c0, 1.0)
            + a1_ref[0].astype(_f32) / jnp.maximum(c1, 1.0))


def _k2a_body(dense_ref, a0_ref, a1_ref, c0_ref, c1_ref, wl_ref, ycat_ref):
    m = _mean(a0_ref, a1_ref, c0_ref, c1_ref)
    h = jax.nn.relu(dense_ref[...] + m)
    ycat_ref[0] = jnp.dot(h, wl_ref[0].T, precision=_HIGH).astype(_bf16)


_agg_spec0 = pl.BlockSpec((1, _BLK, H), lambda i: (0, i % _GRID, 0))
_agg_spec1 = pl.BlockSpec((1, _BLK, H), lambda i: (1, i % _GRID, 0))
_cnt_spec0 = pl.BlockSpec((1, _BLK, CW), lambda i: (0, i % _GRID, 0))
_cnt_spec1 = pl.BlockSpec((1, _BLK, CW), lambda i: (1, i % _GRID, 0))

_k2a = pl.pallas_call(
    _k2a_body,
    grid=(_GRID2,),
    in_specs=[pl.BlockSpec((_BLK, H), lambda i: (i % _GRID, 0)),
              _agg_spec0, _agg_spec1, _cnt_spec0, _cnt_spec1,
              pl.BlockSpec((1, H, H), lambda i: (i // _GRID, 0, 0))],
    out_specs=pl.BlockSpec((1, _BLK, H),
                           lambda i: (i // _GRID, i % _GRID, 0)),
    out_shape=jax.ShapeDtypeStruct((2, NP, H), _bf16),
)


def _k2b_body(dense_ref, a0_ref, a1_ref, c0_ref, c1_ref,
              wr0_ref, wr1_ref, b0_ref, b1_ref, dense1_ref):
    m = _mean(a0_ref, a1_ref, c0_ref, c1_ref)
    h = jax.nn.relu(dense_ref[...] + m)
    dense1_ref[...] = (jnp.dot(h, wr0_ref[...].T) + b0_ref[...]
                       + jnp.dot(h, wr1_ref[...].T) + b1_ref[...])


_k2b = pl.pallas_call(
    _k2b_body,
    grid=(_GRID,),
    in_specs=[pl.BlockSpec((_BLK, H), lambda i: (i, 0)),
              pl.BlockSpec((1, _BLK, H), lambda i: (0, i, 0)),
              pl.BlockSpec((1, _BLK, H), lambda i: (1, i, 0)),
              pl.BlockSpec((1, _BLK, CW), lambda i: (0, i, 0)),
              pl.BlockSpec((1, _BLK, CW), lambda i: (1, i, 0)),
              pl.BlockSpec((H, H), lambda i: (0, 0)),
              pl.BlockSpec((H, H), lambda i: (0, 0)),
              pl.BlockSpec((1, H), lambda i: (0, 0)),
              pl.BlockSpec((1, H), lambda i: (0, 0))],
    out_specs=pl.BlockSpec((_BLK, H), lambda i: (i, 0)),
    out_shape=jax.ShapeDtypeStruct((NP, H), _f32),
)


def _k3_body(dense_ref, a0_ref, a1_ref, c0_ref, c1_ref,
             wcls_ref, bcls_ref, out_ref):
    m = _mean(a0_ref, a1_ref, c0_ref, c1_ref)
    h2 = dense_ref[...] + m
    out_ref[...] = jnp.dot(h2, wcls_ref[...].T) + bcls_ref[...]


_k3 = pl.pallas_call(
    _k3_body,
    grid=(_GRID,),
    in_specs=[pl.BlockSpec((_BLK, H), lambda i: (i, 0)),
              pl.BlockSpec((1, _BLK, H), lambda i: (0, i, 0)),
              pl.BlockSpec((1, _BLK, H), lambda i: (1, i, 0)),
              pl.BlockSpec((1, _BLK, CW), lambda i: (0, i, 0)),
              pl.BlockSpec((1, _BLK, CW), lambda i: (1, i, 0)),
              pl.BlockSpec((C, H), lambda i: (0, 0)),
              pl.BlockSpec((1, C), lambda i: (0, 0))],
    out_specs=pl.BlockSpec((_BLK, C), lambda i: (i, 0)),
    out_shape=jax.ShapeDtypeStruct((NP, C), _f32),
)


def kernel(x, edge_index_rel0, edge_index_rel1,
           Wl_0_0, bl_0_0, Wr_0_0, Wl_0_1, bl_0_1, Wr_0_1,
           Wl_1_0, bl_1_0, Wr_1_0, Wl_1_1, bl_1_1, Wr_1_1,
           W_cls, b_cls):
    e0 = edge_index_rel0.reshape(2, EROWS, SUPER)
    e1 = edge_index_rel1.reshape(2, EROWS, SUPER)

    wl0 = jnp.stack([Wl_0_0, Wl_0_1])
    ycat0 = _k1a(x, wl0)

    agg0, cnt = _make_sc(True)(ycat0, e0, e1)
    dense0 = _k1b(x, Wr_0_0, Wr_0_1,
                  bl_0_0.reshape(1, H), bl_0_1.reshape(1, H))

    wl1 = jnp.stack([Wl_1_0, Wl_1_1])
    ycat1 = _k2a(dense0, agg0, agg0, cnt, cnt, wl1)

    agg1 = _make_sc(False)(ycat1, e0, e1)
    if isinstance(agg1, (list, tuple)):
        agg1 = agg1[0]
    dense1 = _k2b(dense0, agg0, agg0, cnt, cnt, Wr_1_0, Wr_1_1,
                  bl_1_0.reshape(1, H), bl_1_1.reshape(1, H))

    out = _k3(dense1, agg1, agg1, cnt, cnt, W_cls, b_cls.reshape(1, C))
    return out[:N]
